# baseline (speedup 1.0000x reference)
import functools

import jax
import jax.numpy as jnp
from jax import lax
from jax.experimental import pallas as pl
from jax.experimental.pallas import tpu as pltpu
from jax.experimental.pallas import tpu_sc as plsc

_N, _D = 10000, 128
_NC, _NS = 2, 16
_NW = _NC * _NS
_CHUNK = 128
_CPT = 80
_EPAD = _NW * _CPT * _CHUNK
_RPT = 624
_TAIL = _N - _NS * _RPT
_AGG_ROWS = _N + 128
_BN_EPS = 1e-5


def _sc_partials(x, row2d, col2d):
    mesh = plsc.VectorSubcoreMesh(core_axis_name="c", subcore_axis_name="s")

    @functools.partial(
        pl.kernel,
        out_type=jax.ShapeDtypeStruct((_NC, _N, _D), jnp.float32),
        mesh=mesh,
        scratch_types=[
            pltpu.VMEM((_CPT // 2, _CHUNK), jnp.int32),
            pltpu.VMEM((_CPT // 2, _CHUNK), jnp.int32),
            pltpu.VMEM((_CHUNK, _D), jnp.float32),
            pltpu.VMEM((_CHUNK, _D), jnp.float32),
            pltpu.VMEM_SHARED((_AGG_ROWS, _D), jnp.float32),
            pltpu.SemaphoreType.DMA,
            pltpu.SemaphoreType.DMA,
        ],
    )
    def k(x_hbm, row_hbm, col_hbm, out_hbm, idx_r, idx_c, g0, g1, agg, s0, s1):
        c = lax.axis_index("c")
        s = lax.axis_index("s")
        wid = c * _NS + s
        pltpu.sync_copy(x_hbm.at[pl.ds(s * _RPT, _RPT)],
                        agg.at[pl.ds(s * _RPT, _RPT)])

        @pl.when(s < 2)
        def _():
            base = _NS * _RPT + s * 8
            pltpu.sync_copy(x_hbm.at[pl.ds(base, 8)], agg.at[pl.ds(base, 8)])

        plsc.subcore_barrier()

        def pair(i, carry):
            j0 = i * 2
            j1 = j0 + 1
            cp0 = pltpu.async_copy(x_hbm.at[idx_c.at[j0]], g0, s0)
            cp1 = pltpu.async_copy(x_hbm.at[idx_c.at[j1]], g1, s1)
            cp0.wait()
            pltpu.sync_copy(g0, agg.at[idx_r.at[j0]], add=True)
            cp1.wait()
            pltpu.sync_copy(g1, agg.at[idx_r.at[j1]], add=True)
            return carry

        for grp in range(2):
            base = wid * _CPT + grp * (_CPT // 2)
            pltpu.sync_copy(row_hbm.at[pl.ds(base, _CPT // 2)], idx_r)
            pltpu.sync_copy(col_hbm.at[pl.ds(base, _CPT // 2)], idx_c)
            lax.fori_loop(0, _CPT // 4, pair, 0)
        plsc.subcore_barrier()
        pltpu.sync_copy(agg.at[pl.ds(s * _RPT, _RPT)],
                        out_hbm.at[c].at[pl.ds(s * _RPT, _RPT)])

        @pl.when(s < 2)
        def _():
            base = _NS * _RPT + s * 8
            pltpu.sync_copy(agg.at[pl.ds(base, 8)],
                            out_hbm.at[c].at[pl.ds(base, 8)])

    return k(x, row2d, col2d)


def _tc_finish(x, parts, eps11, W1, b1, g1, be1, W2, b2, g2, be2):
    def body(x_ref, p_ref, eps_ref, w1_ref, b1_ref, g1_ref, be1_ref,
             w2_ref, b2_ref, g2_ref, be2_ref, o_ref):
        eps = eps_ref[0, 0]
        h = p_ref[0] + p_ref[1] + (eps - 1.0) * x_ref[...]
        h = jnp.dot(h, w1_ref[...], preferred_element_type=jnp.float32) + b1_ref[...]
        m = jnp.mean(h, axis=0, keepdims=True)
        v = jnp.mean((h - m) * (h - m), axis=0, keepdims=True)
        h = (h - m) * lax.rsqrt(v + _BN_EPS) * g1_ref[...] + be1_ref[...]
        h = jnp.maximum(h, 0.0)
        h = jnp.dot(h, w2_ref[...], preferred_element_type=jnp.float32) + b2_ref[...]
        m = jnp.mean(h, axis=0, keepdims=True)
        v = jnp.mean((h - m) * (h - m), axis=0, keepdims=True)
        h = (h - m) * lax.rsqrt(v + _BN_EPS) * g2_ref[...] + be2_ref[...]
        o_ref[...] = jnp.maximum(h, 0.0)

    return pl.pallas_call(
        body,
        out_shape=jax.ShapeDtypeStruct((_N, _D), jnp.float32),
    )(x, parts, eps11, W1, b1, g1, be1, W2, b2, g2, be2)


def kernel(x, edge_index, eps, W1, b1, g1, be1, W2, b2, g2, be2):
    e = edge_index.shape[1]
    pad = _EPAD - e
    pad_rows = _N + (jnp.arange(pad, dtype=jnp.int32) % 128)
    row = jnp.concatenate([edge_index[0], pad_rows]).reshape(-1, _CHUNK)
    col = jnp.concatenate(
        [edge_index[1], jnp.zeros((pad,), jnp.int32)]).reshape(-1, _CHUNK)
    parts = _sc_partials(x, row, col)
    eps11 = jnp.reshape(eps, (1, 1)).astype(jnp.float32)
    return _tc_finish(
        x, parts, eps11,
        W1, b1.reshape(1, _D), g1.reshape(1, _D), be1.reshape(1, _D),
        W2, b2.reshape(1, _D), g2.reshape(1, _D), be2.reshape(1, _D))


# speedup vs baseline: 1.0514x; 1.0514x over previous
import functools

import jax
import jax.numpy as jnp
from jax import lax
from jax.experimental import pallas as pl
from jax.experimental.pallas import tpu as pltpu
from jax.experimental.pallas import tpu_sc as plsc

_N, _D = 10000, 128
_NC, _NS = 2, 16
_NW = _NC * _NS
_CHUNK = 128
_CPT = 80
_EPAD = _NW * _CPT * _CHUNK
_RPT = 624
_TAIL = _N - _NS * _RPT
_AGG_ROWS = _N + 128
_BN_EPS = 1e-5


def _sc_partials(x, row2d, col2d):
    mesh = plsc.VectorSubcoreMesh(core_axis_name="c", subcore_axis_name="s")

    @functools.partial(
        pl.kernel,
        out_type=jax.ShapeDtypeStruct((_NC, _N, _D), jnp.float32),
        mesh=mesh,
        scratch_types=[
            pltpu.VMEM((_CPT // 2, _CHUNK), jnp.int32),
            pltpu.VMEM((_CPT // 2, _CHUNK), jnp.int32),
            pltpu.VMEM((_CHUNK, _D), jnp.float32),
            pltpu.VMEM((_CHUNK, _D), jnp.float32),
            pltpu.VMEM_SHARED((_AGG_ROWS, _D), jnp.float32),
            pltpu.SemaphoreType.DMA,
            pltpu.SemaphoreType.DMA,
        ],
    )
    def k(x_hbm, row_hbm, col_hbm, out_hbm, idx_r, idx_c, g0, g1, agg, s0, s1):
        c = lax.axis_index("c")
        s = lax.axis_index("s")
        wid = (1 - c) * _NS + s
        pltpu.sync_copy(x_hbm.at[pl.ds(s * _RPT, _RPT)],
                        agg.at[pl.ds(s * _RPT, _RPT)])

        @pl.when(s < 2)
        def _():
            base = _NS * _RPT + s * 8
            pltpu.sync_copy(x_hbm.at[pl.ds(base, 8)], agg.at[pl.ds(base, 8)])

        plsc.subcore_barrier()

        def pair(i, carry):
            j0 = i * 2
            j1 = j0 + 1
            cp0 = pltpu.async_copy(x_hbm.at[idx_c.at[j0]], g0, s0)
            cp1 = pltpu.async_copy(x_hbm.at[idx_c.at[j1]], g1, s1)
            cp0.wait()
            pltpu.sync_copy(g0, agg.at[idx_r.at[j0]], add=True)
            cp1.wait()
            pltpu.sync_copy(g1, agg.at[idx_r.at[j1]], add=True)
            return carry

        for grp in range(2):
            base = wid * _CPT + grp * (_CPT // 2)
            pltpu.sync_copy(row_hbm.at[pl.ds(base, _CPT // 2)], idx_r)
            pltpu.sync_copy(col_hbm.at[pl.ds(base, _CPT // 2)], idx_c)
            lax.fori_loop(0, _CPT // 4, pair, 0)
        plsc.subcore_barrier()
        pltpu.sync_copy(agg.at[pl.ds(s * _RPT, _RPT)],
                        out_hbm.at[c].at[pl.ds(s * _RPT, _RPT)])

        @pl.when(s < 2)
        def _():
            base = _NS * _RPT + s * 8
            pltpu.sync_copy(agg.at[pl.ds(base, 8)],
                            out_hbm.at[c].at[pl.ds(base, 8)])

    return k(x, row2d, col2d)


def _tc_finish(x, parts, eps11, W1, b1, g1, be1, W2, b2, g2, be2):
    def body(x_ref, p_ref, eps_ref, w1_ref, b1_ref, g1_ref, be1_ref,
             w2_ref, b2_ref, g2_ref, be2_ref, o_ref):
        eps = eps_ref[0, 0]
        h = p_ref[0] + p_ref[1] + (eps - 1.0) * x_ref[...]
        h = jnp.dot(h, w1_ref[...], preferred_element_type=jnp.float32) + b1_ref[...]
        m = jnp.mean(h, axis=0, keepdims=True)
        v = jnp.mean((h - m) * (h - m), axis=0, keepdims=True)
        h = (h - m) * lax.rsqrt(v + _BN_EPS) * g1_ref[...] + be1_ref[...]
        h = jnp.maximum(h, 0.0)
        h = jnp.dot(h, w2_ref[...], preferred_element_type=jnp.float32) + b2_ref[...]
        m = jnp.mean(h, axis=0, keepdims=True)
        v = jnp.mean((h - m) * (h - m), axis=0, keepdims=True)
        h = (h - m) * lax.rsqrt(v + _BN_EPS) * g2_ref[...] + be2_ref[...]
        o_ref[...] = jnp.maximum(h, 0.0)

    return pl.pallas_call(
        body,
        out_shape=jax.ShapeDtypeStruct((_N, _D), jnp.float32),
    )(x, parts, eps11, W1, b1, g1, be1, W2, b2, g2, be2)


def kernel(x, edge_index, eps, W1, b1, g1, be1, W2, b2, g2, be2):
    e = edge_index.shape[1]
    pad = _EPAD - e
    pad_rows = _N + (jnp.arange(pad, dtype=jnp.int32) % 128)
    row = jnp.concatenate([edge_index[0], pad_rows]).reshape(-1, _CHUNK)
    col = jnp.concatenate(
        [edge_index[1], jnp.zeros((pad,), jnp.int32)]).reshape(-1, _CHUNK)
    parts = _sc_partials(x, row, col)
    eps11 = jnp.reshape(eps, (1, 1)).astype(jnp.float32)
    return _tc_finish(
        x, parts, eps11,
        W1, b1.reshape(1, _D), g1.reshape(1, _D), be1.reshape(1, _D),
        W2, b2.reshape(1, _D), g2.reshape(1, _D), be2.reshape(1, _D))


# speedup vs baseline: 2.8666x; 2.7265x over previous
import functools

import jax
import jax.numpy as jnp
from jax import lax
from jax.experimental import pallas as pl
from jax.experimental.pallas import tpu as pltpu
from jax.experimental.pallas import tpu_sc as plsc

_N, _D = 10000, 128
_NC, _NS = 2, 16
_NW = _NC * _NS
_CHUNK = 128
_CPT = 80
_EPAD = _NW * _CPT * _CHUNK
_RPT = 624
_TAIL = _N - _NS * _RPT
_AGG_ROWS = _N + 128
_BN_EPS = 1e-5


def _sc_partials(x, row2d, col2d):
    mesh = plsc.VectorSubcoreMesh(core_axis_name="c", subcore_axis_name="s")

    @functools.partial(
        pl.kernel,
        out_type=jax.ShapeDtypeStruct((_NC, _N, _D), jnp.float32),
        mesh=mesh,
        scratch_types=[
            pltpu.VMEM((_CPT // 2, _CHUNK), jnp.int32),
            pltpu.VMEM((_CPT // 2, _CHUNK), jnp.int32),
            pltpu.VMEM((_CHUNK, _D), jnp.float32),
            pltpu.VMEM((_CHUNK, _D), jnp.float32),
            pltpu.VMEM_SHARED((_AGG_ROWS, _D), jnp.float32),
            pltpu.SemaphoreType.DMA,
            pltpu.SemaphoreType.DMA,
        ],
    )
    def k(x_hbm, row_hbm, col_hbm, out_hbm, idx_r, idx_c, g0, g1, agg, s0, s1):
        c = lax.axis_index("c")
        s = lax.axis_index("s")
        wid = c * _NS + s
        pltpu.sync_copy(x_hbm.at[pl.ds(s * _RPT, _RPT)],
                        agg.at[pl.ds(s * _RPT, _RPT)])

        @pl.when(s < 2)
        def _():
            base = _NS * _RPT + s * 8
            pltpu.sync_copy(x_hbm.at[pl.ds(base, 8)], agg.at[pl.ds(base, 8)])

        plsc.subcore_barrier()

        def pair(i, carry):
            j0 = i * 2
            j1 = j0 + 1
            cp0 = pltpu.async_copy(x_hbm.at[idx_c.at[j0]], g0, s0)
            cp1 = pltpu.async_copy(x_hbm.at[idx_c.at[j1]], g1, s1)
            cp0.wait()
            pltpu.sync_copy(g0, agg.at[idx_r.at[j0]], add=True)
            cp1.wait()
            pltpu.sync_copy(g1, agg.at[idx_r.at[j1]], add=True)
            return carry

        for grp in range(2):
            base = wid * _CPT + grp * (_CPT // 2)
            pltpu.sync_copy(row_hbm.at[pl.ds(base, _CPT // 2)], idx_r)
            pltpu.sync_copy(col_hbm.at[pl.ds(base, _CPT // 2)], idx_c)
            lax.fori_loop(0, _CPT // 4, pair, 0)
        plsc.subcore_barrier()
        pltpu.sync_copy(agg.at[pl.ds(s * _RPT, _RPT)],
                        out_hbm.at[c].at[pl.ds(s * _RPT, _RPT)])

        @pl.when(s < 2)
        def _():
            base = _NS * _RPT + s * 8
            pltpu.sync_copy(agg.at[pl.ds(base, 8)],
                            out_hbm.at[c].at[pl.ds(base, 8)])

    return k(x, row2d, col2d)


def _tc_finish(x, parts, eps11, W1, b1, g1, be1, W2, b2, g2, be2):
    def body(x_ref, p_ref, eps_ref, w1_ref, b1_ref, g1_ref, be1_ref,
             w2_ref, b2_ref, g2_ref, be2_ref, o_ref):
        eps = eps_ref[0, 0]
        h = p_ref[0] + p_ref[1] + (eps - 1.0) * x_ref[...]
        h = jnp.dot(h, w1_ref[...], preferred_element_type=jnp.float32) + b1_ref[...]
        m = jnp.mean(h, axis=0, keepdims=True)
        v = jnp.mean((h - m) * (h - m), axis=0, keepdims=True)
        h = (h - m) * lax.rsqrt(v + _BN_EPS) * g1_ref[...] + be1_ref[...]
        h = jnp.maximum(h, 0.0)
        h = jnp.dot(h, w2_ref[...], preferred_element_type=jnp.float32) + b2_ref[...]
        m = jnp.mean(h, axis=0, keepdims=True)
        v = jnp.mean((h - m) * (h - m), axis=0, keepdims=True)
        h = (h - m) * lax.rsqrt(v + _BN_EPS) * g2_ref[...] + be2_ref[...]
        o_ref[...] = jnp.maximum(h, 0.0)

    return pl.pallas_call(
        body,
        out_shape=jax.ShapeDtypeStruct((_N, _D), jnp.float32),
    )(x, parts, eps11, W1, b1, g1, be1, W2, b2, g2, be2)


def kernel(x, edge_index, eps, W1, b1, g1, be1, W2, b2, g2, be2):
    e = edge_index.shape[1]
    pad = _EPAD - e
    pad_rows = _N + (jnp.arange(pad, dtype=jnp.int32) % 128)
    row = jnp.concatenate([edge_index[0], pad_rows]).reshape(-1, _CHUNK)
    pad_cols = jnp.arange(pad, dtype=jnp.int32) % _N
    col = jnp.concatenate([edge_index[1], pad_cols]).reshape(-1, _CHUNK)
    parts = _sc_partials(x, row, col)
    eps11 = jnp.reshape(eps, (1, 1)).astype(jnp.float32)
    return _tc_finish(
        x, parts, eps11,
        W1, b1.reshape(1, _D), g1.reshape(1, _D), be1.reshape(1, _D),
        W2, b2.reshape(1, _D), g2.reshape(1, _D), be2.reshape(1, _D))


# speedup vs baseline: 2.9387x; 1.0251x over previous
import functools

import jax
import jax.numpy as jnp
from jax import lax
from jax.experimental import pallas as pl
from jax.experimental.pallas import tpu as pltpu
from jax.experimental.pallas import tpu_sc as plsc

_N, _D = 10000, 128
_DH = _D // 2
_NC, _NS = 2, 16
_CHUNK = 128
_CPT = 160
_SLAB = 40
_EPAD = _NS * _CPT * _CHUNK
_RPT = 624
_AGG_ROWS = _N + 128
_BN_EPS = 1e-5


def _sc_agg(x01, row2d, col2d):
    mesh = plsc.VectorSubcoreMesh(core_axis_name="c", subcore_axis_name="s")

    @functools.partial(
        pl.kernel,
        out_type=jax.ShapeDtypeStruct((_NC, _N, _DH), jnp.float32),
        mesh=mesh,
        compiler_params=pltpu.CompilerParams(use_tc_tiling_on_sc=False),
        scratch_types=[
            pltpu.VMEM((_SLAB, _CHUNK), jnp.int32),
            pltpu.VMEM((_SLAB, _CHUNK), jnp.int32),
            pltpu.VMEM((_CHUNK, _DH), jnp.float32),
            pltpu.VMEM((_CHUNK, _DH), jnp.float32),
            pltpu.VMEM((_CHUNK, _DH), jnp.float32),
            pltpu.VMEM((_CHUNK, _DH), jnp.float32),
            pltpu.VMEM_SHARED((_AGG_ROWS, _DH), jnp.float32),
            pltpu.SemaphoreType.DMA,
            pltpu.SemaphoreType.DMA,
            pltpu.SemaphoreType.DMA,
            pltpu.SemaphoreType.DMA,
            pltpu.SemaphoreType.DMA,
            pltpu.SemaphoreType.DMA,
            pltpu.SemaphoreType.DMA,
            pltpu.SemaphoreType.DMA,
        ],
    )
    def k(x_hbm, row_hbm, col_hbm, out_hbm,
          idx_r, idx_c, b0, b1, b2, b3, agg,
          g0, g1, g2, g3, s0, s1, s2, s3):
        c = lax.axis_index("c")
        s = lax.axis_index("s")
        xh = x_hbm.at[c]
        gb = (b0, b1, b2, b3)
        gs = (g0, g1, g2, g3)
        ss = (s0, s1, s2, s3)

        pltpu.sync_copy(xh.at[pl.ds(s * _RPT, _RPT)],
                        agg.at[pl.ds(s * _RPT, _RPT)])

        @pl.when(s < 2)
        def _():
            base = _NS * _RPT + s * 8
            pltpu.sync_copy(xh.at[pl.ds(base, 8)], agg.at[pl.ds(base, 8)])

        plsc.subcore_barrier()

        def G(j, k_):
            pltpu.async_copy(xh.at[idx_c.at[j]], gb[k_], gs[k_])

        def S(j, k_):
            pltpu.async_copy(gb[k_], agg.at[idx_r.at[j]], ss[k_], add=True)

        def Wg(k_):
            pltpu.make_async_copy(xh.at[pl.ds(0, _CHUNK)], gb[k_], gs[k_]).wait()

        def Ws(k_):
            pltpu.make_async_copy(gb[k_], agg.at[pl.ds(0, _CHUNK)], ss[k_]).wait()

        for slab in range(_CPT // _SLAB):
            base = s * _CPT + slab * _SLAB
            pltpu.sync_copy(row_hbm.at[pl.ds(base, _SLAB)], idx_r)
            pltpu.sync_copy(col_hbm.at[pl.ds(base, _SLAB)], idx_c)
            G(0, 0)
            G(1, 1)
            Wg(0)
            S(0, 0)
            G(2, 2)
            Wg(1)
            S(1, 1)
            G(3, 3)

            def steady(g, carry):
                for k_ in range(4):
                    j = 4 + g * 4 + k_
                    Ws(k_)
                    Wg((k_ + 2) % 4)
                    S(j - 2, (k_ + 2) % 4)
                    G(j, k_)
                return carry

            lax.fori_loop(0, (_SLAB - 4) // 4, steady, 0)
            Wg(2)
            S(_SLAB - 2, 2)
            Wg(3)
            S(_SLAB - 1, 3)
            Ws(0)
            Ws(1)
            Ws(2)
            Ws(3)

        plsc.subcore_barrier()
        pltpu.sync_copy(agg.at[pl.ds(s * _RPT, _RPT)],
                        out_hbm.at[c].at[pl.ds(s * _RPT, _RPT)])

        @pl.when(s < 2)
        def _():
            base = _NS * _RPT + s * 8
            pltpu.sync_copy(agg.at[pl.ds(base, 8)],
                            out_hbm.at[c].at[pl.ds(base, 8)])

    return k(x01, row2d, col2d)


def _tc_finish(x, parts, eps11, W1, b1, g1, be1, W2, b2, g2, be2):
    def body(x_ref, p_ref, eps_ref, w1_ref, b1_ref, g1_ref, be1_ref,
             w2_ref, b2_ref, g2_ref, be2_ref, o_ref):
        eps = eps_ref[0, 0]
        h = eps * x_ref[...] + jnp.concatenate([p_ref[0], p_ref[1]], axis=-1)
        h = jnp.dot(h, w1_ref[...], preferred_element_type=jnp.float32) + b1_ref[...]
        m = jnp.mean(h, axis=0, keepdims=True)
        v = jnp.mean((h - m) * (h - m), axis=0, keepdims=True)
        h = (h - m) * lax.rsqrt(v + _BN_EPS) * g1_ref[...] + be1_ref[...]
        h = jnp.maximum(h, 0.0)
        h = jnp.dot(h, w2_ref[...], preferred_element_type=jnp.float32) + b2_ref[...]
        m = jnp.mean(h, axis=0, keepdims=True)
        v = jnp.mean((h - m) * (h - m), axis=0, keepdims=True)
        h = (h - m) * lax.rsqrt(v + _BN_EPS) * g2_ref[...] + be2_ref[...]
        o_ref[...] = jnp.maximum(h, 0.0)

    return pl.pallas_call(
        body,
        out_shape=jax.ShapeDtypeStruct((_N, _D), jnp.float32),
    )(x, parts, eps11, W1, b1, g1, be1, W2, b2, g2, be2)


def kernel(x, edge_index, eps, W1, b1, g1, be1, W2, b2, g2, be2):
    e = edge_index.shape[1]
    pad = _EPAD - e
    pad_rows = _N + (jnp.arange(pad, dtype=jnp.int32) % 128)
    row = jnp.concatenate([edge_index[0], pad_rows]).reshape(-1, _CHUNK)
    pad_cols = jnp.arange(pad, dtype=jnp.int32) % _N
    col = jnp.concatenate([edge_index[1], pad_cols]).reshape(-1, _CHUNK)
    x01 = jnp.stack([x[:, :_DH], x[:, _DH:]])
    parts = _sc_agg(x01, row, col)
    eps11 = jnp.reshape(eps, (1, 1)).astype(jnp.float32)
    return _tc_finish(
        x, parts, eps11,
        W1, b1.reshape(1, _D), g1.reshape(1, _D), be1.reshape(1, _D),
        W2, b2.reshape(1, _D), g2.reshape(1, _D), be2.reshape(1, _D))


# speedup vs baseline: 3.2638x; 1.1106x over previous
import functools

import jax
import jax.numpy as jnp
from jax import lax
from jax.experimental import pallas as pl
from jax.experimental.pallas import tpu as pltpu
from jax.experimental.pallas import tpu_sc as plsc

_N, _D = 10000, 128
_DH = _D // 2
_NC, _NS = 2, 16
_CHUNK = 128
_NCH = 2500
_CPT = _NCH // _NS
_RPT = 624
_AGG_ROWS = _N + 8
_BN_EPS = 1e-5


def _sc_agg(x01, row2d, col2d):
    mesh = plsc.VectorSubcoreMesh(core_axis_name="c", subcore_axis_name="s")

    @functools.partial(
        pl.kernel,
        out_type=jax.ShapeDtypeStruct((_N, _D), jnp.float32),
        mesh=mesh,
        compiler_params=pltpu.CompilerParams(use_tc_tiling_on_sc=False),
        scratch_types=[
            pltpu.VMEM((_CPT, _CHUNK), jnp.int32),
            pltpu.VMEM((_CPT, _CHUNK), jnp.int32),
            pltpu.VMEM((1, _CHUNK), jnp.int32),
            pltpu.VMEM((1, _CHUNK), jnp.int32),
            pltpu.VMEM((_CHUNK, _DH), jnp.float32),
            pltpu.VMEM((_CHUNK, _DH), jnp.float32),
            pltpu.VMEM((_CHUNK, _DH), jnp.float32),
            pltpu.VMEM((_CHUNK, _DH), jnp.float32),
            pltpu.VMEM_SHARED((_AGG_ROWS, _DH), jnp.float32),
            pltpu.SemaphoreType.DMA,
            pltpu.SemaphoreType.DMA,
            pltpu.SemaphoreType.DMA,
            pltpu.SemaphoreType.DMA,
            pltpu.SemaphoreType.DMA,
            pltpu.SemaphoreType.DMA,
            pltpu.SemaphoreType.DMA,
            pltpu.SemaphoreType.DMA,
        ],
    )
    def k(x_hbm, row_hbm, col_hbm, out_hbm,
          idx_r, idx_c, lx_r, lx_c, b0, b1, b2, b3, agg,
          g0, g1, g2, g3, s0, s1, s2, s3):
        c = lax.axis_index("c")
        s = lax.axis_index("s")
        xh = x_hbm.at[c]
        gb = (b0, b1, b2, b3)
        gs = (g0, g1, g2, g3)
        ss = (s0, s1, s2, s3)

        pltpu.sync_copy(row_hbm.at[pl.ds(s * _CPT, _CPT)], idx_r)
        pltpu.sync_copy(col_hbm.at[pl.ds(s * _CPT, _CPT)], idx_c)

        @pl.when(s < 4)
        def _():
            base = _NS * _CPT + s
            pltpu.sync_copy(row_hbm.at[pl.ds(base, 1)], lx_r)
            pltpu.sync_copy(col_hbm.at[pl.ds(base, 1)], lx_c)

        pltpu.sync_copy(xh.at[pl.ds(s * _RPT, _RPT)],
                        agg.at[pl.ds(s * _RPT, _RPT)])

        @pl.when(s < 2)
        def _():
            base = _NS * _RPT + s * 8
            pltpu.sync_copy(xh.at[pl.ds(base, 8)], agg.at[pl.ds(base, 8)])

        plsc.subcore_barrier()

        def G(j, k_):
            pltpu.async_copy(xh.at[idx_c.at[j]], gb[k_], gs[k_])

        def S(j, k_):
            pltpu.async_copy(gb[k_], agg.at[idx_r.at[j]], ss[k_], add=True)

        def Wg(k_):
            pltpu.make_async_copy(xh.at[pl.ds(0, _CHUNK)], gb[k_], gs[k_]).wait()

        def Ws(k_):
            pltpu.make_async_copy(gb[k_], agg.at[pl.ds(0, _CHUNK)], ss[k_]).wait()

        G(0, 0)
        G(1, 1)
        Wg(0)
        S(0, 0)
        G(2, 2)
        Wg(1)
        S(1, 1)
        G(3, 3)

        def steady(g, carry):
            for k_ in range(4):
                j = 4 + g * 4 + k_
                Ws(k_)
                Wg((k_ + 2) % 4)
                S(j - 2, (k_ + 2) % 4)
                G(j, k_)
            return carry

        lax.fori_loop(0, (_CPT - 4) // 4, steady, 0)
        Wg(2)
        S(_CPT - 2, 2)
        Wg(3)
        S(_CPT - 1, 3)
        Ws(0)
        Ws(1)
        Ws(2)
        Ws(3)

        @pl.when(s < 4)
        def _():
            pltpu.async_copy(xh.at[lx_c.at[0]], b0, g0).wait()
            pltpu.sync_copy(b0, agg.at[lx_r.at[0]], add=True)

        plsc.subcore_barrier()
        pltpu.sync_copy(agg.at[pl.ds(s * _RPT, _RPT)],
                        out_hbm.at[pl.ds(s * _RPT, _RPT), pl.ds(c * _DH, _DH)])

        @pl.when(s < 2)
        def _():
            base = _NS * _RPT + s * 8
            pltpu.sync_copy(agg.at[pl.ds(base, 8)],
                            out_hbm.at[pl.ds(base, 8), pl.ds(c * _DH, _DH)])

    return k(x01, row2d, col2d)


def _tc_finish(x, p, eps11, W1, b1, g1, be1, W2, b2, g2, be2):
    def body(x_ref, p_ref, eps_ref, w1_ref, b1_ref, g1_ref, be1_ref,
             w2_ref, b2_ref, g2_ref, be2_ref, o_ref):
        eps = eps_ref[0, 0]
        h = eps * x_ref[...] + p_ref[...]
        h = jnp.dot(h, w1_ref[...], preferred_element_type=jnp.float32) + b1_ref[...]
        m = jnp.mean(h, axis=0, keepdims=True)
        v = jnp.mean((h - m) * (h - m), axis=0, keepdims=True)
        h = (h - m) * lax.rsqrt(v + _BN_EPS) * g1_ref[...] + be1_ref[...]
        h = jnp.maximum(h, 0.0)
        h = jnp.dot(h, w2_ref[...], preferred_element_type=jnp.float32) + b2_ref[...]
        m = jnp.mean(h, axis=0, keepdims=True)
        v = jnp.mean((h - m) * (h - m), axis=0, keepdims=True)
        h = (h - m) * lax.rsqrt(v + _BN_EPS) * g2_ref[...] + be2_ref[...]
        o_ref[...] = jnp.maximum(h, 0.0)

    return pl.pallas_call(
        body,
        out_shape=jax.ShapeDtypeStruct((_N, _D), jnp.float32),
    )(x, p, eps11, W1, b1, g1, be1, W2, b2, g2, be2)


def kernel(x, edge_index, eps, W1, b1, g1, be1, W2, b2, g2, be2):
    row = edge_index[0].reshape(_NCH, _CHUNK)
    col = edge_index[1].reshape(_NCH, _CHUNK)
    x01 = jnp.stack([x[:, :_DH], x[:, _DH:]])
    p = _sc_agg(x01, row, col)
    eps11 = jnp.reshape(eps, (1, 1)).astype(jnp.float32)
    return _tc_finish(
        x, p, eps11,
        W1, b1.reshape(1, _D), g1.reshape(1, _D), be1.reshape(1, _D),
        W2, b2.reshape(1, _D), g2.reshape(1, _D), be2.reshape(1, _D))


# speedup vs baseline: 3.4222x; 1.0485x over previous
import functools

import jax
import jax.numpy as jnp
from jax import lax
from jax.experimental import pallas as pl
from jax.experimental.pallas import tpu as pltpu
from jax.experimental.pallas import tpu_sc as plsc

_N, _D = 10000, 128
_DH = _D // 2
_NC, _NS = 2, 16
_CHUNK = 128
_NCH = 2496
_CPT = _NCH // _NS
_RPT = 624
_PAGG = _N // 2
_BN_EPS = 1e-5


def _sc_agg(x01p, eidx, lidx):
    mesh = plsc.VectorSubcoreMesh(core_axis_name="c", subcore_axis_name="s")

    @functools.partial(
        pl.kernel,
        out_type=jax.ShapeDtypeStruct((_N, _D), jnp.float32),
        mesh=mesh,
        compiler_params=pltpu.CompilerParams(use_tc_tiling_on_sc=False),
        scratch_types=[
            pltpu.VMEM((_CPT, _CHUNK), jnp.int32),
            pltpu.VMEM((_CPT, _CHUNK), jnp.int32),
            pltpu.VMEM((1, _CHUNK), jnp.int32),
            pltpu.VMEM((1, _CHUNK), jnp.int32),
            pltpu.VMEM((_CHUNK, _DH), jnp.float32),
            pltpu.VMEM((_CHUNK, _DH), jnp.float32),
            pltpu.VMEM((_CHUNK, _DH), jnp.float32),
            pltpu.VMEM((_CHUNK, _DH), jnp.float32),
            pltpu.VMEM_SHARED((_N, _DH), jnp.float32),
            pltpu.SemaphoreType.DMA,
            pltpu.SemaphoreType.DMA,
            pltpu.SemaphoreType.DMA,
            pltpu.SemaphoreType.DMA,
            pltpu.SemaphoreType.DMA,
            pltpu.SemaphoreType.DMA,
            pltpu.SemaphoreType.DMA,
            pltpu.SemaphoreType.DMA,
        ],
    )
    def k(x_hbm, eidx_hbm, lidx_hbm, out_hbm,
          idx_r, idx_c, lx_r, lx_c, b0, b1, b2, b3, agg,
          g0, g1, g2, g3, s0, s1, s2, s3):
        c = lax.axis_index("c")
        s = lax.axis_index("s")
        xh = x_hbm.at[c]
        aggv = agg
        gb = (b0, b1, b2, b3)
        gs = (g0, g1, g2, g3)
        ss = (s0, s1, s2, s3)

        pltpu.sync_copy(eidx_hbm.at[0].at[pl.ds(s * _CPT, _CPT)], idx_r)
        pltpu.sync_copy(eidx_hbm.at[1].at[pl.ds(s * _CPT, _CPT)], idx_c)

        @pl.when(s < 4)
        def _():
            pltpu.sync_copy(lidx_hbm.at[0].at[pl.ds(s, 1)], lx_r)
            pltpu.sync_copy(lidx_hbm.at[1].at[pl.ds(s, 1)], lx_c)

        pltpu.sync_copy(xh.at[pl.ds(s * _RPT, _RPT)],
                        agg.at[pl.ds(s * _RPT, _RPT)])

        @pl.when(s < 2)
        def _():
            base = _NS * _RPT + s * 8
            pltpu.sync_copy(xh.at[pl.ds(base, 8)], agg.at[pl.ds(base, 8)])

        plsc.subcore_barrier()

        def G(j, k_):
            pltpu.async_copy(xh.at[idx_c.at[j]], gb[k_], gs[k_])

        def S(j, k_):
            pltpu.async_copy(gb[k_], aggv.at[idx_r.at[j]], ss[k_], add=True)

        def Wg(k_):
            pltpu.make_async_copy(xh.at[pl.ds(0, _CHUNK)], gb[k_], gs[k_]).wait()

        def Ws(k_):
            pltpu.make_async_copy(gb[k_], aggv.at[pl.ds(0, _CHUNK)], ss[k_]).wait()

        G(0, 0)
        G(1, 1)
        Wg(0)
        S(0, 0)
        G(2, 2)
        Wg(1)
        S(1, 1)
        G(3, 3)

        def steady(g, carry):
            for k_ in range(4):
                j = 4 + g * 4 + k_
                Ws(k_)
                Wg((k_ + 2) % 4)
                S(j - 2, (k_ + 2) % 4)
                G(j, k_)
            return carry

        lax.fori_loop(0, (_CPT - 4) // 4, steady, 0)
        Wg(2)
        S(_CPT - 2, 2)
        Wg(3)
        S(_CPT - 1, 3)
        Ws(0)
        Ws(1)
        Ws(2)
        Ws(3)

        @pl.when(s < 4)
        def _():
            pltpu.async_copy(xh.at[lx_c.at[0]], b0, g0).wait()
            pltpu.sync_copy(b0, aggv.at[lx_r.at[0]], add=True)

        plsc.subcore_barrier()
        pltpu.sync_copy(aggv.at[pl.ds(s * _RPT, _RPT)],
                        out_hbm.at[pl.ds(s * _RPT, _RPT), pl.ds(c * _DH, _DH)])

        @pl.when(s < 2)
        def _():
            base = _NS * _RPT + s * 8
            pltpu.sync_copy(aggv.at[pl.ds(base, 8)],
                            out_hbm.at[pl.ds(base, 8), pl.ds(c * _DH, _DH)])

    return k(x01p, eidx, lidx)


def _tc_finish(x, p, eps11, W1, b1, g1, be1, W2, b2, g2, be2):
    def body(x_ref, p_ref, eps_ref, w1_ref, b1_ref, g1_ref, be1_ref,
             w2_ref, b2_ref, g2_ref, be2_ref, o_ref):
        eps = eps_ref[0, 0]
        h = eps * x_ref[...] + p_ref[...]
        h = jnp.dot(h, w1_ref[...], preferred_element_type=jnp.float32) + b1_ref[...]
        m = jnp.mean(h, axis=0, keepdims=True)
        v = jnp.mean((h - m) * (h - m), axis=0, keepdims=True)
        h = (h - m) * lax.rsqrt(v + _BN_EPS) * g1_ref[...] + be1_ref[...]
        h = jnp.maximum(h, 0.0)
        h = jnp.dot(h, w2_ref[...], preferred_element_type=jnp.float32) + b2_ref[...]
        m = jnp.mean(h, axis=0, keepdims=True)
        v = jnp.mean((h - m) * (h - m), axis=0, keepdims=True)
        h = (h - m) * lax.rsqrt(v + _BN_EPS) * g2_ref[...] + be2_ref[...]
        o_ref[...] = jnp.maximum(h, 0.0)

    return pl.pallas_call(
        body,
        out_shape=jax.ShapeDtypeStruct((_N, _D), jnp.float32),
    )(x, p, eps11, W1, b1, g1, be1, W2, b2, g2, be2)


def kernel(x, edge_index, eps, W1, b1, g1, be1, W2, b2, g2, be2):
    em = edge_index[:, :_NCH * _CHUNK].reshape(2, _NCH, _CHUNK)
    lm = edge_index[:, _NCH * _CHUNK:].reshape(2, 4, _CHUNK)
    x01p = jnp.stack([x[:, :_DH], x[:, _DH:]])
    p = _sc_agg(x01p, em, lm)
    eps11 = jnp.reshape(eps, (1, 1)).astype(jnp.float32)
    return _tc_finish(
        x, p, eps11,
        W1, b1.reshape(1, _D), g1.reshape(1, _D), be1.reshape(1, _D),
        W2, b2.reshape(1, _D), g2.reshape(1, _D), be2.reshape(1, _D))
